# baseline (speedup 1.0000x reference)
import functools

import jax
import jax.numpy as jnp
from jax import lax
from jax.experimental import pallas as pl
from jax.experimental.pallas import tpu as pltpu
from jax.experimental.pallas import tpu_sc as plsc

_NC = 2
_NS = 16
_L = 16
_R = 49920
_CK = 2048
_DH = 4
_U = 8


def kernel(text_indices, context_indices, w_in, w_out):
    B = text_indices.shape[0]
    N, E = w_in.shape
    assert E == _NC * _NS and B % _CK == 0
    tail = N - 2 * _R
    assert 0 < tail <= 256
    nck = B // _CK
    b_per_t = B // _NS
    mesh = plsc.VectorSubcoreMesh(core_axis_name="c", subcore_axis_name="s")

    @functools.partial(
        pl.kernel,
        mesh=mesh,
        out_type=jax.ShapeDtypeStruct((_NC, B), jnp.float32),
        compiler_params=pltpu.CompilerParams(needs_layout_passes=False),
        scratch_types=[
            pltpu.VMEM((_R + tail,), jnp.float32),
            pltpu.VMEM((_R + tail,), jnp.float32),
            pltpu.VMEM((tail,), jnp.float32),
            pltpu.VMEM((_CK,), jnp.int32),
            pltpu.VMEM((_CK,), jnp.int32),
            pltpu.VMEM((B,), jnp.float32),
            pltpu.VMEM((_CK,), jnp.float32),
            pltpu.VMEM((_CK,), jnp.float32),
            pltpu.VMEM((_DH, B // _NS), jnp.float32),
            pltpu.HBM((_NC, _NS, B), jnp.float32),
            pltpu.SemaphoreType.DMA,
            pltpu.SemaphoreType.DMA,
            pltpu.SemaphoreType.DMA,
            pltpu.SemaphoreType.DMA,
        ],
    )
    def sc_kernel(ti_hbm, ci_hbm, wt_in_hbm, wt_out_hbm, out_hbm,
                  row_a, row_b, tail_v, idx_v0, idx_v1, xfull, pc_v0, pc_v1,
                  pbuf, p_hbm, sem_a, sem_b, sem_i, sem_p):
        c = lax.axis_index("c")
        t = lax.axis_index("s")
        d = c * _NS + t

        def stage(tbl_hbm):
            return (pltpu.async_copy(tbl_hbm.at[d, pl.ds(0, _R)],
                                     row_a.at[pl.ds(0, _R)], sem_a),
                    pltpu.async_copy(tbl_hbm.at[d, pl.ds(_R, _R)],
                                     row_b.at[pl.ds(0, _R)], sem_b),
                    pltpu.async_copy(tbl_hbm.at[d, pl.ds(2 * _R, tail)],
                                     tail_v, sem_b))

        def wait_and_merge(cps):
            for cp in cps:
                cp.wait()
            for k in range(tail // _L):
                row_b[pl.ds(_R + k * _L, _L)] = tail_v[pl.ds(k * _L, _L)]

        def gather2(iv):
            m0 = iv < _R
            g0 = plsc.load_gather(row_a, [iv], mask=m0)
            m1 = iv >= _R
            g1 = plsc.load_gather(row_b, [iv - _R], mask=m1)
            return jnp.where(m0, g0, 0.0) + jnp.where(m1, g1, 0.0)

        idxbufs = (idx_v0, idx_v1)
        pcbufs = (pc_v0, pc_v1)

        def idx_fetch(i_hbm, k):
            return pltpu.async_copy(i_hbm.at[pl.ds(k * _CK, _CK)],
                                    idxbufs[k % 2], sem_i)

        cps = stage(wt_in_hbm)
        icp = idx_fetch(ti_hbm, 0)
        wait_and_merge(cps)
        for k in range(nck):
            icp.wait()
            icp = idx_fetch(ti_hbm, k + 1) if k + 1 < nck else None
            kbase = k * _CK
            ibuf = idxbufs[k % 2]

            @plsc.parallel_loop(0, _CK, step=_L, unroll=_U)
            def xbody(i):
                xfull[pl.ds(kbase + i, _L)] = gather2(ibuf[pl.ds(i, _L)])

        cps = stage(wt_out_hbm)
        icp = idx_fetch(ci_hbm, 0)
        wait_and_merge(cps)
        pushes = [None, None]
        for k in range(nck):
            icp.wait()
            icp = idx_fetch(ci_hbm, k + 1) if k + 1 < nck else None
            kbase = k * _CK
            ibuf = idxbufs[k % 2]
            pbuf2 = pcbufs[k % 2]
            if pushes[k % 2] is not None:
                pushes[k % 2].wait()

            @plsc.parallel_loop(0, _CK, step=_L, unroll=_U)
            def cbody(i):
                cv = gather2(ibuf[pl.ds(i, _L)])
                pbuf2[pl.ds(i, _L)] = cv * xfull[pl.ds(kbase + i, _L)]

            pushes[k % 2] = pltpu.async_copy(
                pbuf2, p_hbm.at[c, t, pl.ds(kbase, _CK)], sem_p)
        for push in pushes:
            if push is not None:
                push.wait()

        plsc.subcore_barrier()

        bbase = t * b_per_t
        for dchunk in range(_NS // _DH):
            dbase = dchunk * _DH
            pltpu.sync_copy(
                p_hbm.at[c, pl.ds(dbase, _DH), pl.ds(bbase, b_per_t)], pbuf)

            @plsc.parallel_loop(0, b_per_t, step=_L, unroll=_U)
            def rbody(v):
                sl = pl.ds(v, _L)
                acc = pbuf[0, sl]
                for dd in range(1, _DH):
                    acc = acc + pbuf[dd, sl]
                if dchunk:
                    acc = acc + xfull[sl]
                xfull[sl] = acc

        pltpu.sync_copy(xfull.at[pl.ds(0, b_per_t)],
                        out_hbm.at[c, pl.ds(bbase, b_per_t)])

    partials = sc_kernel(text_indices, context_indices, w_in.T, w_out.T)
    return partials[0] + partials[1]


# speedup vs baseline: 1.0143x; 1.0143x over previous
import functools

import jax
import jax.numpy as jnp
from jax import lax
from jax.experimental import pallas as pl
from jax.experimental.pallas import tpu as pltpu
from jax.experimental.pallas import tpu_sc as plsc

_NC = 2
_NS = 16
_L = 16
_R = 49920
_CK = 2048
_DH = 4
_U = 4


def kernel(text_indices, context_indices, w_in, w_out):
    B = text_indices.shape[0]
    N, E = w_in.shape
    assert E == _NC * _NS and B % _CK == 0
    tail = N - 2 * _R
    assert 0 < tail <= 256
    nck = B // _CK
    b_per_t = B // _NS
    mesh = plsc.VectorSubcoreMesh(core_axis_name="c", subcore_axis_name="s")

    @functools.partial(
        pl.kernel,
        mesh=mesh,
        out_type=jax.ShapeDtypeStruct((_NC, B), jnp.float32),
        compiler_params=pltpu.CompilerParams(needs_layout_passes=False),
        scratch_types=[
            pltpu.VMEM((_R + tail,), jnp.float32),
            pltpu.VMEM((_R + tail,), jnp.float32),
            pltpu.VMEM((tail,), jnp.float32),
            pltpu.VMEM((_CK,), jnp.int32),
            pltpu.VMEM((_CK,), jnp.int32),
            pltpu.VMEM((B,), jnp.float32),
            pltpu.VMEM((_CK,), jnp.float32),
            pltpu.VMEM((_CK,), jnp.float32),
            pltpu.VMEM((_DH, B // _NS), jnp.float32),
            pltpu.HBM((_NC, _NS, B), jnp.float32),
            pltpu.SemaphoreType.DMA,
            pltpu.SemaphoreType.DMA,
            pltpu.SemaphoreType.DMA,
            pltpu.SemaphoreType.DMA,
        ],
    )
    def sc_kernel(ti_hbm, ci_hbm, wt_in_hbm, wt_out_hbm, out_hbm,
                  row_a, row_b, tail_v, idx_v0, idx_v1, xfull, pc_v0, pc_v1,
                  pbuf, p_hbm, sem_a, sem_b, sem_i, sem_p):
        c = lax.axis_index("c")
        t = lax.axis_index("s")
        d = c * _NS + t

        def stage(tbl_hbm):
            return (pltpu.async_copy(tbl_hbm.at[d, pl.ds(0, _R)],
                                     row_a.at[pl.ds(0, _R)], sem_a),
                    pltpu.async_copy(tbl_hbm.at[d, pl.ds(_R, _R)],
                                     row_b.at[pl.ds(0, _R)], sem_b),
                    pltpu.async_copy(tbl_hbm.at[d, pl.ds(2 * _R, tail)],
                                     tail_v, sem_b))

        def wait_and_merge(cps):
            for cp in cps:
                cp.wait()
            for k in range(tail // _L):
                row_b[pl.ds(_R + k * _L, _L)] = tail_v[pl.ds(k * _L, _L)]

        def gather2(iv):
            m0 = iv < _R
            g0 = plsc.load_gather(row_a, [iv], mask=m0)
            m1 = iv >= _R
            g1 = plsc.load_gather(row_b, [iv - _R], mask=m1)
            return jnp.where(m0, g0, 0.0) + jnp.where(m1, g1, 0.0)

        idxbufs = (idx_v0, idx_v1)
        pcbufs = (pc_v0, pc_v1)

        def idx_fetch(i_hbm, k):
            return pltpu.async_copy(i_hbm.at[pl.ds(k * _CK, _CK)],
                                    idxbufs[k % 2], sem_i)

        sched = ([(ti_hbm, k) for k in range(nck)] * 2
                 + [(ci_hbm, k) for k in range(nck)])
        fetches = [None] * len(sched)

        def fetch(j):
            hbm, k = sched[j]
            fetches[j] = pltpu.async_copy(
                hbm.at[pl.ds(k * _CK, _CK)], idxbufs[j % 2], sem_i)

        cps_a = [pltpu.async_copy(wt_in_hbm.at[d, pl.ds(0, _R)],
                                  row_a.at[pl.ds(0, _R)], sem_a)]
        cps_b = [pltpu.async_copy(wt_in_hbm.at[d, pl.ds(_R, _R)],
                                  row_b.at[pl.ds(0, _R)], sem_b),
                 pltpu.async_copy(wt_in_hbm.at[d, pl.ds(2 * _R, tail)],
                                  tail_v, sem_b)]
        fetch(0)
        for cp in cps_a:
            cp.wait()
        for k in range(nck):
            j = k
            fetch(j + 1)
            fetches[j].wait()
            kbase = k * _CK
            ibuf = idxbufs[j % 2]

            @plsc.parallel_loop(0, _CK, step=_L, unroll=_U)
            def xbody0(i):
                iv = ibuf[pl.ds(i, _L)]
                m = iv < _R
                gv = plsc.load_gather(row_a, [iv], mask=m)
                xfull[pl.ds(kbase + i, _L)] = jnp.where(m, gv, 0.0)

        for cp in cps_b:
            cp.wait()
        for k in range(tail // _L):
            row_b[pl.ds(_R + k * _L, _L)] = tail_v[pl.ds(k * _L, _L)]
        cps_ca = [pltpu.async_copy(wt_out_hbm.at[d, pl.ds(0, _R)],
                                   row_a.at[pl.ds(0, _R)], sem_a)]
        for k in range(nck):
            j = nck + k
            if j + 1 < len(sched):
                fetch(j + 1)
            fetches[j].wait()
            kbase = k * _CK
            ibuf = idxbufs[j % 2]

            @plsc.parallel_loop(0, _CK, step=_L, unroll=_U)
            def xbody1(i):
                iv = ibuf[pl.ds(i, _L)]
                m = iv >= _R
                gv = plsc.load_gather(row_b, [iv - _R], mask=m)
                xfull[pl.ds(kbase + i, _L)] = (
                    xfull[pl.ds(kbase + i, _L)] + jnp.where(m, gv, 0.0))

        cps_cb = [pltpu.async_copy(wt_out_hbm.at[d, pl.ds(_R, _R)],
                                   row_b.at[pl.ds(0, _R)], sem_b),
                  pltpu.async_copy(wt_out_hbm.at[d, pl.ds(2 * _R, tail)],
                                   tail_v, sem_b)]
        for cp in cps_ca:
            cp.wait()
        for cp in cps_cb:
            cp.wait()
        for k in range(tail // _L):
            row_b[pl.ds(_R + k * _L, _L)] = tail_v[pl.ds(k * _L, _L)]

        pushes = [None, None]
        for k in range(nck):
            j = 2 * nck + k
            if j + 1 < len(sched):
                fetch(j + 1)
            fetches[j].wait()
            kbase = k * _CK
            ibuf = idxbufs[j % 2]
            pbuf2 = pcbufs[k % 2]
            if pushes[k % 2] is not None:
                pushes[k % 2].wait()

            @plsc.parallel_loop(0, _CK, step=_L, unroll=_U)
            def cbody(i):
                cv = gather2(ibuf[pl.ds(i, _L)])
                pbuf2[pl.ds(i, _L)] = cv * xfull[pl.ds(kbase + i, _L)]

            pushes[k % 2] = pltpu.async_copy(
                pbuf2, p_hbm.at[c, t, pl.ds(kbase, _CK)], sem_p)
        for push in pushes:
            if push is not None:
                push.wait()

        plsc.subcore_barrier()

        bbase = t * b_per_t
        for dchunk in range(_NS // _DH):
            dbase = dchunk * _DH
            pltpu.sync_copy(
                p_hbm.at[c, pl.ds(dbase, _DH), pl.ds(bbase, b_per_t)], pbuf)

            @plsc.parallel_loop(0, b_per_t, step=_L, unroll=_U)
            def rbody(v):
                sl = pl.ds(v, _L)
                acc = pbuf[0, sl]
                for dd in range(1, _DH):
                    acc = acc + pbuf[dd, sl]
                if dchunk:
                    acc = acc + xfull[sl]
                xfull[sl] = acc

        pltpu.sync_copy(xfull.at[pl.ds(0, b_per_t)],
                        out_hbm.at[c, pl.ds(bbase, b_per_t)])

    partials = sc_kernel(text_indices, context_indices, w_in.T, w_out.T)
    return partials[0] + partials[1]


# speedup vs baseline: 1.0305x; 1.0159x over previous
import functools

import jax
import jax.numpy as jnp
from jax import lax
from jax.experimental import pallas as pl
from jax.experimental.pallas import tpu as pltpu
from jax.experimental.pallas import tpu_sc as plsc

_NC = 2
_NS = 16
_L = 16
_R = 49920
_CK = 2048
_DH = 2
_U = 4


def kernel(text_indices, context_indices, w_in, w_out):
    B = text_indices.shape[0]
    N, E = w_in.shape
    assert E == _NC * _NS and B % _CK == 0
    tail = N - 2 * _R
    assert 0 < tail <= 256
    nck = B // _CK
    b_per_t = B // _NS
    mesh = plsc.VectorSubcoreMesh(core_axis_name="c", subcore_axis_name="s")

    @functools.partial(
        pl.kernel,
        mesh=mesh,
        out_type=jax.ShapeDtypeStruct((_NC, B), jnp.float32),
        compiler_params=pltpu.CompilerParams(needs_layout_passes=False),
        scratch_types=[
            pltpu.VMEM((_R + tail,), jnp.float32),
            pltpu.VMEM((_R + tail,), jnp.float32),
            pltpu.VMEM((tail,), jnp.float32),
            pltpu.VMEM((_CK,), jnp.int32),
            pltpu.VMEM((_CK,), jnp.int32),
            pltpu.VMEM((B,), jnp.float32),
            pltpu.VMEM((_CK,), jnp.float32),
            pltpu.VMEM((_CK,), jnp.float32),
            pltpu.VMEM((_DH, B // _NS), jnp.float32),
            pltpu.VMEM((_DH, B // _NS), jnp.float32),
            pltpu.HBM((_NC, _NS, B), jnp.float32),
            pltpu.SemaphoreType.DMA,
            pltpu.SemaphoreType.DMA,
            pltpu.SemaphoreType.DMA,
            pltpu.SemaphoreType.DMA,
        ],
    )
    def sc_kernel(ti_hbm, ci_hbm, wt_in_hbm, wt_out_hbm, out_hbm,
                  row_a, row_b, tail_v, idx_v0, idx_v1, xfull, pc_v0, pc_v1,
                  pbuf, pbuf2_b, p_hbm, sem_a, sem_b, sem_i, sem_p):
        c = lax.axis_index("c")
        t = lax.axis_index("s")
        d = c * _NS + t

        def stage(tbl_hbm):
            return (pltpu.async_copy(tbl_hbm.at[d, pl.ds(0, _R)],
                                     row_a.at[pl.ds(0, _R)], sem_a),
                    pltpu.async_copy(tbl_hbm.at[d, pl.ds(_R, _R)],
                                     row_b.at[pl.ds(0, _R)], sem_b),
                    pltpu.async_copy(tbl_hbm.at[d, pl.ds(2 * _R, tail)],
                                     tail_v, sem_b))

        def wait_and_merge(cps):
            for cp in cps:
                cp.wait()
            for k in range(tail // _L):
                row_b[pl.ds(_R + k * _L, _L)] = tail_v[pl.ds(k * _L, _L)]

        def gather2(iv):
            m0 = iv < _R
            g0 = plsc.load_gather(row_a, [iv], mask=m0)
            m1 = iv >= _R
            g1 = plsc.load_gather(row_b, [iv - _R], mask=m1)
            return jnp.where(m0, g0, 0.0) + jnp.where(m1, g1, 0.0)

        idxbufs = (idx_v0, idx_v1)
        pcbufs = (pc_v0, pc_v1)

        def idx_fetch(i_hbm, k):
            return pltpu.async_copy(i_hbm.at[pl.ds(k * _CK, _CK)],
                                    idxbufs[k % 2], sem_i)

        cps = stage(wt_in_hbm)
        icp = idx_fetch(ti_hbm, 0)
        wait_and_merge(cps)
        for k in range(nck):
            icp.wait()
            icp = idx_fetch(ti_hbm, k + 1) if k + 1 < nck else None
            kbase = k * _CK
            ibuf = idxbufs[k % 2]

            @plsc.parallel_loop(0, _CK, step=_L, unroll=_U)
            def xbody(i):
                xfull[pl.ds(kbase + i, _L)] = gather2(ibuf[pl.ds(i, _L)])

        cps = stage(wt_out_hbm)
        icp = idx_fetch(ci_hbm, 0)
        wait_and_merge(cps)
        pushes = [None, None]
        for k in range(nck):
            icp.wait()
            icp = idx_fetch(ci_hbm, k + 1) if k + 1 < nck else None
            kbase = k * _CK
            ibuf = idxbufs[k % 2]
            pbuf2 = pcbufs[k % 2]
            if pushes[k % 2] is not None:
                pushes[k % 2].wait()

            @plsc.parallel_loop(0, _CK, step=_L, unroll=_U)
            def cbody(i):
                cv = gather2(ibuf[pl.ds(i, _L)])
                pbuf2[pl.ds(i, _L)] = cv * xfull[pl.ds(kbase + i, _L)]

            pushes[k % 2] = pltpu.async_copy(
                pbuf2, p_hbm.at[c, t, pl.ds(kbase, _CK)], sem_p)
        for push in pushes:
            if push is not None:
                push.wait()

        plsc.subcore_barrier()

        bbase = t * b_per_t
        nd = _NS // _DH

        def pull(dchunk, buf):
            return pltpu.async_copy(
                p_hbm.at[c, pl.ds(dchunk * _DH, _DH), pl.ds(bbase, b_per_t)],
                buf, sem_i)

        pulls = [pull(0, pbuf), pull(1, pbuf2_b)]
        for dchunk in range(nd):
            pb = (pbuf, pbuf2_b)[dchunk % 2]
            pulls[dchunk % 2].wait()

            @plsc.parallel_loop(0, b_per_t, step=_L, unroll=_U)
            def rbody(v):
                sl = pl.ds(v, _L)
                acc = pb[0, sl]
                for dd in range(1, _DH):
                    acc = acc + pb[dd, sl]
                if dchunk:
                    acc = acc + xfull[sl]
                xfull[sl] = acc

            if dchunk + 2 < nd:
                pulls[dchunk % 2] = pull(dchunk + 2, pb)

        pltpu.sync_copy(xfull.at[pl.ds(0, b_per_t)],
                        out_hbm.at[c, pl.ds(bbase, b_per_t)])

    partials = sc_kernel(text_indices, context_indices, w_in.T, w_out.T)
    return partials[0] + partials[1]


# speedup vs baseline: 1.0334x; 1.0028x over previous
import functools

import jax
import jax.numpy as jnp
from jax import lax
from jax.experimental import pallas as pl
from jax.experimental.pallas import tpu as pltpu
from jax.experimental.pallas import tpu_sc as plsc

_NC = 2
_NS = 16
_L = 16
_R = 49920
_DH = 4
_U = 4


def kernel(text_indices, context_indices, w_in, w_out):
    B = text_indices.shape[0]
    N, E = w_in.shape
    assert E == _NC * _NS and B % (_NS * _L * _U) == 0
    tail = N - 2 * _R
    assert 0 < tail <= 256
    half = B // 2
    b_per_t = B // _NS
    mesh = plsc.VectorSubcoreMesh(core_axis_name="c", subcore_axis_name="s")

    @functools.partial(
        pl.kernel,
        mesh=mesh,
        out_type=jax.ShapeDtypeStruct((_NC, B), jnp.float32),
        compiler_params=pltpu.CompilerParams(needs_layout_passes=False),
        scratch_types=[
            pltpu.VMEM((_R + tail,), jnp.float32),
            pltpu.VMEM((tail,), jnp.float32),
            pltpu.VMEM((B,), jnp.int32),
            pltpu.VMEM((half,), jnp.int32),
            pltpu.VMEM((B,), jnp.float32),
            pltpu.VMEM((B,), jnp.float32),
            pltpu.VMEM((_DH, B // _NS), jnp.float32),
            pltpu.VMEM_SHARED((_NS, B), jnp.float32),
        ],
    )
    def sc_kernel(ti_hbm, ci_hbm, wt_in_hbm, wt_out_hbm, out_hbm,
                  row_v, tail_v, ti_v, cic_v, xfull, pc_v, pbuf, p_sp):
        c = lax.axis_index("c")
        t = lax.axis_index("s")
        d = c * _NS + t

        pltpu.sync_copy(ti_hbm, ti_v)

        for r in range(2):
          with jax.named_scope(f"xstage{r}"):
            if r == 0:
                pltpu.sync_copy(wt_in_hbm.at[d, pl.ds(0, _R)],
                                row_v.at[pl.ds(0, _R)])
            else:
                pltpu.sync_copy(wt_in_hbm.at[d, pl.ds(_R, _R)],
                                row_v.at[pl.ds(0, _R)])
                pltpu.sync_copy(wt_in_hbm.at[d, pl.ds(2 * _R, tail)], tail_v)
                for k in range(tail // _L):
                    row_v[pl.ds(_R + k * _L, _L)] = tail_v[pl.ds(k * _L, _L)]

          with jax.named_scope(f"xgather{r}"):
            @plsc.parallel_loop(0, B, step=_L, unroll=_U)
            def xbody(i):
                sl = pl.ds(i, _L)
                iv = ti_v[sl]
                if r == 0:
                    m = iv < _R
                    gv = plsc.load_gather(row_v, [iv], mask=m)
                    xfull[sl] = jnp.where(m, gv, 0.0)
                else:
                    m = iv >= _R
                    gv = plsc.load_gather(row_v, [iv - _R], mask=m)
                    xfull[sl] = xfull[sl] + jnp.where(m, gv, 0.0)

        for r in range(2):
          with jax.named_scope(f"cstage{r}"):
            if r == 0:
                pltpu.sync_copy(wt_out_hbm.at[d, pl.ds(0, _R)],
                                row_v.at[pl.ds(0, _R)])
            else:
                pltpu.sync_copy(wt_out_hbm.at[d, pl.ds(_R, _R)],
                                row_v.at[pl.ds(0, _R)])
                pltpu.sync_copy(wt_out_hbm.at[d, pl.ds(2 * _R, tail)], tail_v)
                for k in range(tail // _L):
                    row_v[pl.ds(_R + k * _L, _L)] = tail_v[pl.ds(k * _L, _L)]
          for ih in range(2):
            with jax.named_scope(f"cgather{r}_{ih}"):
                hbase = ih * half
                pltpu.sync_copy(ci_hbm.at[pl.ds(hbase, half)], cic_v)

                @plsc.parallel_loop(0, half, step=_L, unroll=_U)
                def cbody(o):
                    sl = pl.ds(hbase + o, _L)
                    iv = cic_v[pl.ds(o, _L)]
                    if r == 0:
                        m = iv < _R
                        gv = plsc.load_gather(row_v, [iv], mask=m)
                        pc_v[sl] = jnp.where(m, gv, 0.0) * xfull[sl]
                    else:
                        m = iv >= _R
                        gv = plsc.load_gather(row_v, [iv - _R], mask=m)
                        pc_v[sl] = (pc_v[sl]
                                    + jnp.where(m, gv, 0.0) * xfull[sl])

        with jax.named_scope("push_barrier"):
            pltpu.sync_copy(pc_v, p_sp.at[t])
            plsc.subcore_barrier()

        bbase = t * b_per_t
        for dchunk in range(_NS // _DH):
          with jax.named_scope(f"phaseB{dchunk}"):
            dbase = dchunk * _DH
            pltpu.sync_copy(p_sp.at[pl.ds(dbase, _DH), pl.ds(bbase, b_per_t)],
                            pbuf)

            @plsc.parallel_loop(0, b_per_t, step=_L, unroll=_U)
            def rbody(v):
                sl = pl.ds(v, _L)
                acc = pbuf[0, sl]
                for dd in range(1, _DH):
                    acc = acc + pbuf[dd, sl]
                if dchunk:
                    acc = acc + pc_v[sl]
                pc_v[sl] = acc
        pltpu.sync_copy(pc_v.at[pl.ds(0, b_per_t)],
                        out_hbm.at[c, pl.ds(bbase, b_per_t)])

    partials = sc_kernel(text_indices, context_indices, w_in.T, w_out.T)
    return partials[0] + partials[1]


# speedup vs baseline: 1.0375x; 1.0040x over previous
import functools

import jax
import jax.numpy as jnp
from jax import lax
from jax.experimental import pallas as pl
from jax.experimental.pallas import tpu as pltpu
from jax.experimental.pallas import tpu_sc as plsc

_NC = 2
_NS = 16
_L = 16
_R = 49920
_DH = 4
_U = 4


def kernel(text_indices, context_indices, w_in, w_out):
    B = text_indices.shape[0]
    N, E = w_in.shape
    assert E == _NC * _NS and B % (_NS * _L * _U) == 0
    tail = N - 2 * _R
    assert 0 < tail <= 256
    half = B // 2
    b_per_t = B // _NS
    mesh = plsc.VectorSubcoreMesh(core_axis_name="c", subcore_axis_name="s")

    @functools.partial(
        pl.kernel,
        mesh=mesh,
        out_type=jax.ShapeDtypeStruct((_NC, B), jnp.float32),
        compiler_params=pltpu.CompilerParams(needs_layout_passes=False),
        scratch_types=[
            pltpu.VMEM((_R + tail,), jnp.float32),
            pltpu.VMEM((tail,), jnp.float32),
            pltpu.VMEM((B,), jnp.int32),
            pltpu.VMEM((half,), jnp.int32),
            pltpu.VMEM((B,), jnp.float32),
            pltpu.VMEM((B,), jnp.float32),
            pltpu.VMEM((_DH, B // _NS), jnp.float32),
            pltpu.VMEM_SHARED((_NS, B), jnp.float32),
        ],
    )
    def sc_kernel(ti_hbm, ci_hbm, wt_in_hbm, wt_out_hbm, out_hbm,
                  row_v, tail_v, ti_v, cic_v, xfull, pc_v, pbuf, p_sp):
        c = lax.axis_index("c")
        t = lax.axis_index("s")
        d = c * _NS + t

        pltpu.sync_copy(ti_hbm, ti_v)

        def stage_row(tbl_hbm, r):
            if r == 0:
                pltpu.sync_copy(tbl_hbm.at[d, pl.ds(0, _R)],
                                row_v.at[pl.ds(0, _R)])
            else:
                pltpu.sync_copy(tbl_hbm.at[d, pl.ds(_R, _R)],
                                row_v.at[pl.ds(0, _R)])
                pltpu.sync_copy(tbl_hbm.at[d, pl.ds(2 * _R, tail)], tail_v)
                for k in range(tail // _L):
                    row_v[pl.ds(_R + k * _L, _L)] = tail_v[pl.ds(k * _L, _L)]

        for r in range(2):
            stage_row(wt_in_hbm, r)

            @plsc.parallel_loop(0, B, step=_L, unroll=_U)
            def xbody(i):
                sl = pl.ds(i, _L)
                iv = ti_v[sl]
                if r == 0:
                    m = iv < _R
                    gv = plsc.load_gather(row_v, [iv], mask=m)
                    xfull[sl] = jnp.where(m, gv, 0.0)
                else:
                    m = iv >= _R
                    gv = plsc.load_gather(row_v, [iv - _R], mask=m)
                    xfull[sl] = xfull[sl] + jnp.where(m, gv, 0.0)

        for r in range(2):
            stage_row(wt_out_hbm, r)
            for ih in range(2):
                hbase = ih * half
                pltpu.sync_copy(ci_hbm.at[pl.ds(hbase, half)], cic_v)

                @plsc.parallel_loop(0, half, step=_L, unroll=_U)
                def cbody(o):
                    sl = pl.ds(hbase + o, _L)
                    iv = cic_v[pl.ds(o, _L)]
                    if r == 0:
                        m = iv < _R
                        gv = plsc.load_gather(row_v, [iv], mask=m)
                        pc_v[sl] = jnp.where(m, gv, 0.0) * xfull[sl]
                    else:
                        m = iv >= _R
                        gv = plsc.load_gather(row_v, [iv - _R], mask=m)
                        pc_v[sl] = (pc_v[sl]
                                    + jnp.where(m, gv, 0.0) * xfull[sl])

        pltpu.sync_copy(pc_v, p_sp.at[t])
        plsc.subcore_barrier()

        bbase = t * b_per_t
        for dchunk in range(_NS // _DH):
            dbase = dchunk * _DH
            pltpu.sync_copy(p_sp.at[pl.ds(dbase, _DH), pl.ds(bbase, b_per_t)],
                            pbuf)

            @plsc.parallel_loop(0, b_per_t, step=_L, unroll=_U)
            def rbody(v):
                sl = pl.ds(v, _L)
                acc = pbuf[0, sl]
                for dd in range(1, _DH):
                    acc = acc + pbuf[dd, sl]
                if dchunk:
                    acc = acc + pc_v[sl]
                pc_v[sl] = acc

        pltpu.sync_copy(pc_v.at[pl.ds(0, b_per_t)],
                        out_hbm.at[c, pl.ds(bbase, b_per_t)])

    partials = sc_kernel(text_indices, context_indices, w_in.T, w_out.T)
    return partials[0] + partials[1]
